# Initial kernel scaffold; baseline (speedup 1.0000x reference)
import jax
import jax.numpy as jnp
from jax.experimental import pallas as pl


def kernel(queries, keys, values, attn_mask, random_index):
    raise NotImplementedError("write your pallas kernel here")


# speedup vs baseline: 2.6235x; 2.6235x over previous
import functools
import math

import jax
import jax.numpy as jnp
from jax.experimental import pallas as pl
from jax.experimental.pallas import tpu as pltpu

TQ = 256


def _score_body(idx_ref, q_ref, k_ref, m_ref, *, n_heads, l_k, sample_k):
    idx = idx_ref[...]
    j_iota = jax.lax.broadcasted_iota(jnp.int32, (TQ, l_k), 1)
    s_tile = jnp.zeros((TQ, l_k), dtype=jnp.float32)
    for s in range(sample_k):
        col = idx[:, s:s + 1]
        s_tile = s_tile + (j_iota == col).astype(jnp.float32)
    mask = s_tile > 0.0

    def head(h, _):
        q = q_ref[h]
        k = k_ref[h]
        qk = jax.lax.dot_general(q, k, (((1,), (1,)), ((), ())),
                                 preferred_element_type=jnp.float32)
        mx = jnp.max(jnp.where(mask, qk, -jnp.inf), axis=1)
        sm = jnp.sum(qk * s_tile, axis=1)
        m_ref[h, :] = mx - sm / l_k
        return 0

    jax.lax.fori_loop(0, n_heads, head, 0)


def _attend_body(m_ref, q_ref, k_ref, v_ref, ctx_ref, idx_smem, qred_ref,
                 *, l_q, l_k, d, n_top):
    m = m_ref[0, :, :]
    iota = jax.lax.broadcasted_iota(jnp.int32, (1, l_q), 1)

    def sel(i, m):
        mx = jnp.max(m)
        idx = jnp.min(jnp.where(m == mx, iota, l_q))
        idx_smem[i] = idx
        qred_ref[pl.ds(i, 1), :] = q_ref[0, pl.ds(idx, 1), :]
        return jnp.where(iota == idx, -jnp.inf, m)

    jax.lax.fori_loop(0, n_top, sel, m)

    k = k_ref[0]
    v = v_ref[0]
    qred = qred_ref[...]
    scores = jax.lax.dot_general(qred, k, (((1,), (1,)), ((), ())),
                                 preferred_element_type=jnp.float32)
    scores = scores * (1.0 / math.sqrt(d))
    smx = jnp.max(scores, axis=1, keepdims=True)
    e = jnp.exp(scores - smx)
    attn = e / jnp.sum(e, axis=1, keepdims=True)
    upd = jax.lax.dot_general(attn, v, (((1,), (0,)), ((), ())),
                              preferred_element_type=jnp.float32)
    vmean = jnp.mean(v, axis=0, keepdims=True)
    ctx_ref[0] = jnp.broadcast_to(vmean, (l_q, d))
    qred_ref[...] = upd

    def scat(i, _):
        ctx_ref[0, pl.ds(idx_smem[i], 1), :] = qred_ref[pl.ds(i, 1), :]
        return 0

    jax.lax.fori_loop(0, n_top, scat, 0)


def kernel(queries, keys, values, attn_mask, random_index):
    del attn_mask
    b, l_q, h, d = queries.shape
    l_k = keys.shape[1]
    sample_k = random_index.shape[1]
    n_top = min(5 * int(math.ceil(math.log(l_q))), l_q)

    qh = jnp.transpose(queries[0], (1, 0, 2))
    kh = jnp.transpose(keys[0], (1, 0, 2))
    vh = jnp.transpose(values[0], (1, 0, 2))

    n_tiles = l_q // TQ
    m_scores = pl.pallas_call(
        functools.partial(_score_body, n_heads=h, l_k=l_k, sample_k=sample_k),
        grid=(n_tiles,),
        in_specs=[
            pl.BlockSpec((TQ, sample_k), lambda i: (i, 0)),
            pl.BlockSpec((h, TQ, d), lambda i: (0, i, 0)),
            pl.BlockSpec((h, l_k, d), lambda i: (0, 0, 0)),
        ],
        out_specs=pl.BlockSpec((h, TQ), lambda i: (0, i)),
        out_shape=jax.ShapeDtypeStruct((h, l_q), jnp.float32),
    )(random_index, qh, kh)

    m3 = m_scores.reshape(h, 1, l_q)
    ctx = pl.pallas_call(
        functools.partial(_attend_body, l_q=l_q, l_k=l_k, d=d, n_top=n_top),
        grid=(h,),
        in_specs=[
            pl.BlockSpec((1, 1, l_q), lambda i: (i, 0, 0)),
            pl.BlockSpec((1, l_q, d), lambda i: (i, 0, 0)),
            pl.BlockSpec((1, l_k, d), lambda i: (i, 0, 0)),
            pl.BlockSpec((1, l_k, d), lambda i: (i, 0, 0)),
        ],
        out_specs=pl.BlockSpec((1, l_q, d), lambda i: (i, 0, 0)),
        out_shape=jax.ShapeDtypeStruct((h, l_q, d), jnp.float32),
        scratch_shapes=[
            pltpu.SMEM((n_top,), jnp.int32),
            pltpu.VMEM((n_top, d), jnp.float32),
        ],
    )(m3, qh, kh, vh)

    return jnp.transpose(ctx, (1, 0, 2))[None]


# speedup vs baseline: 2.6374x; 1.0053x over previous
import functools
import math

import jax
import jax.numpy as jnp
from jax import lax
from jax.experimental import pallas as pl
from jax.experimental.pallas import tpu as pltpu
from jax.experimental.pallas import tpu_sc as plsc

TQ = 256
SC_CHUNK = 32


def _sbuild_body(idx_hbm, s_hbm, idx_v, rows_v, *, l_k, sample_k, rows_per_w,
                 n_cores):
    wid = lax.axis_index("s") * n_cores + lax.axis_index("c")
    row0 = wid * rows_per_w
    nvec = SC_CHUNK * l_k // 16

    def zero(i, _):
        rows_v[pl.ds(i * 16, 16)] = jnp.zeros((16,), jnp.float32)
        return 0

    lax.fori_loop(0, nvec, zero, 0)

    lanes = lax.iota(jnp.int32, 16)
    ones = jnp.ones((16,), jnp.float32)
    n_full = sample_k // 16
    tail = sample_k - n_full * 16

    for chunk in range(rows_per_w // SC_CHUNK):
        base = row0 + chunk * SC_CHUNK
        pltpu.sync_copy(idx_hbm.at[pl.ds(base * sample_k, SC_CHUNK * sample_k)],
                        idx_v)
        for sgn in (1.0, -1.0):
            for r in range(SC_CHUNK):
                for v in range(n_full + (1 if tail else 0)):
                    if v < n_full:
                        off, mask = v * 16, lanes >= 0
                    else:
                        off, mask = sample_k - 16, lanes >= 16 - tail
                    cols = idx_v[pl.ds(r * sample_k + off, 16)]
                    flat = cols + (r * l_k)
                    plsc.addupdate_scatter(rows_v, [flat], sgn * ones,
                                           mask=mask)
            if sgn == 1.0:
                pltpu.sync_copy(rows_v,
                                s_hbm.at[pl.ds(base * l_k, SC_CHUNK * l_k)])


def _build_s(random_index, l_q, l_k):
    sample_k = random_index.shape[1]
    info = plsc.get_sparse_core_info()
    n_workers = info.num_cores * info.num_subcores
    rows_per_w = l_q // n_workers
    mesh = plsc.VectorSubcoreMesh(core_axis_name="c", subcore_axis_name="s")
    k = pl.kernel(
        functools.partial(_sbuild_body, l_k=l_k, sample_k=sample_k,
                          rows_per_w=rows_per_w, n_cores=info.num_cores),
        mesh=mesh,
        out_type=jax.ShapeDtypeStruct((l_q * l_k,), jnp.float32),
        compiler_params=pltpu.CompilerParams(use_tc_tiling_on_sc=False,
                                             needs_layout_passes=False),
        scratch_types=[
            pltpu.VMEM((SC_CHUNK * sample_k,), jnp.int32),
            pltpu.VMEM((SC_CHUNK * l_k,), jnp.float32),
        ],
    )
    return k(random_index.reshape(-1)).reshape(l_q, l_k)


def _score_body(s_ref, q_ref, k_ref, m_ref, *, n_heads, l_k):
    s_tile = s_ref[...]
    neg = jnp.where(s_tile > 0.0, 0.0, -jnp.inf)

    def head(h, _):
        q = q_ref[h]
        k = k_ref[h]
        qk = lax.dot_general(q, k, (((1,), (1,)), ((), ())),
                             preferred_element_type=jnp.float32)
        ksum = lax.dot_general(s_tile, k, (((1,), (0,)), ((), ())),
                               preferred_element_type=jnp.float32)
        mx = jnp.max(qk + neg, axis=1)
        sm = jnp.sum(q * ksum, axis=1)
        m_ref[h, :] = mx - sm / l_k
        return 0

    lax.fori_loop(0, n_heads, head, 0)


def _attend_body(m_ref, q_ref, k_ref, v_ref, ctx_ref, idx_smem, qred_ref,
                 *, l_q, l_k, d, n_top):
    m = m_ref[0, :, :]
    iota = lax.broadcasted_iota(jnp.int32, (1, l_q), 1)

    def sel(i, m):
        mx = jnp.max(m)
        idx = jnp.min(jnp.where(m == mx, iota, l_q))
        idx_smem[i] = idx
        qred_ref[pl.ds(i, 1), :] = q_ref[0, pl.ds(idx, 1), :]
        return jnp.where(iota == idx, -jnp.inf, m)

    lax.fori_loop(0, n_top, sel, m)

    k = k_ref[0]
    v = v_ref[0]
    qred = qred_ref[...]
    scores = lax.dot_general(qred, k, (((1,), (1,)), ((), ())),
                             preferred_element_type=jnp.float32)
    scores = scores * (1.0 / math.sqrt(d))
    smx = jnp.max(scores, axis=1, keepdims=True)
    e = jnp.exp(scores - smx)
    attn = e / jnp.sum(e, axis=1, keepdims=True)
    upd = lax.dot_general(attn, v, (((1,), (0,)), ((), ())),
                          preferred_element_type=jnp.float32)
    vmean = jnp.mean(v, axis=0, keepdims=True)
    ctx_ref[0] = jnp.broadcast_to(vmean, (l_q, d))
    qred_ref[...] = upd

    def scat(i, _):
        ctx_ref[0, pl.ds(idx_smem[i], 1), :] = qred_ref[pl.ds(i, 1), :]
        return 0

    lax.fori_loop(0, n_top, scat, 0)


def kernel(queries, keys, values, attn_mask, random_index):
    del attn_mask
    b, l_q, h, d = queries.shape
    l_k = keys.shape[1]
    n_top = min(5 * int(math.ceil(math.log(l_q))), l_q)

    s_mat = _build_s(random_index, l_q, l_k)

    qh = jnp.transpose(queries[0], (1, 0, 2))
    kh = jnp.transpose(keys[0], (1, 0, 2))
    vh = jnp.transpose(values[0], (1, 0, 2))

    n_tiles = l_q // TQ
    m_scores = pl.pallas_call(
        functools.partial(_score_body, n_heads=h, l_k=l_k),
        grid=(n_tiles,),
        in_specs=[
            pl.BlockSpec((TQ, l_k), lambda i: (i, 0)),
            pl.BlockSpec((h, TQ, d), lambda i: (0, i, 0)),
            pl.BlockSpec((h, l_k, d), lambda i: (0, 0, 0)),
        ],
        out_specs=pl.BlockSpec((h, TQ), lambda i: (0, i)),
        out_shape=jax.ShapeDtypeStruct((h, l_q), jnp.float32),
    )(s_mat, qh, kh)

    m3 = m_scores.reshape(h, 1, l_q)
    ctx = pl.pallas_call(
        functools.partial(_attend_body, l_q=l_q, l_k=l_k, d=d, n_top=n_top),
        grid=(h,),
        in_specs=[
            pl.BlockSpec((1, 1, l_q), lambda i: (i, 0, 0)),
            pl.BlockSpec((1, l_q, d), lambda i: (i, 0, 0)),
            pl.BlockSpec((1, l_k, d), lambda i: (i, 0, 0)),
            pl.BlockSpec((1, l_k, d), lambda i: (i, 0, 0)),
        ],
        out_specs=pl.BlockSpec((1, l_q, d), lambda i: (i, 0, 0)),
        out_shape=jax.ShapeDtypeStruct((h, l_q, d), jnp.float32),
        scratch_shapes=[
            pltpu.SMEM((n_top,), jnp.int32),
            pltpu.VMEM((n_top, d), jnp.float32),
        ],
    )(m3, qh, kh, vh)

    return jnp.transpose(ctx, (1, 0, 2))[None]
